# Initial kernel scaffold; baseline (speedup 1.0000x reference)
import jax
import jax.numpy as jnp
from jax.experimental import pallas as pl


def kernel(x, edge_index, edge_attr, W1_l, b1, W1_r, W2_l, b2, W2_r, W3_l, b3, W3_r):
    raise NotImplementedError("write your pallas kernel here")


# speedup vs baseline: 5.8884x; 5.8884x over previous
import functools

import jax
import jax.numpy as jnp
from jax import lax
from jax.experimental import pallas as pl
from jax.experimental.pallas import tpu as pltpu
from jax.experimental.pallas import tpu_sc as plsc

_NC = 2
_NS = 16
_C = 2000


def _pad16(n):
    m = _NS * 32
    return ((n + m - 1) // m) * m


@functools.lru_cache(maxsize=None)
def _make_seg_sum_slabs(n_nodes, n_edges, n_slabs):
    assert n_slabs % 2 == 0
    s_half = n_slabs // 2
    n_pad = _pad16(n_nodes)
    half = n_pad // 2
    stripe = half // _NS
    ept = n_edges // _NS
    assert n_edges % (_NS * _C) == 0
    n_chunks = ept // _C
    zrows = stripe // 2
    acc_rows = half + 64

    mesh = plsc.VectorSubcoreMesh(core_axis_name="c", subcore_axis_name="s")

    @functools.partial(
        pl.kernel,
        mesh=mesh,
        compiler_params=pltpu.CompilerParams(use_tc_tiling_on_sc=False),
        out_type=jax.ShapeDtypeStruct((n_slabs, n_pad, 16), jnp.float32),
        scratch_types=[
            pltpu.VMEM((_C,), jnp.int32),
            pltpu.VMEM((_C,), jnp.int32),
            pltpu.VMEM((_C,), jnp.int32),
            pltpu.VMEM((_C,), jnp.int32),
            pltpu.VMEM((_C, 16), jnp.float32),
            pltpu.VMEM_SHARED((acc_rows, 16), jnp.float32),
            pltpu.SemaphoreType.DMA,
        ],
    )
    def kern(table_hbm, src_hbm, dst_hbm, out_hbm,
             idx_v, sidx_v, dst_v, dstm_v, rows_v, acc, sem):
        cid = lax.axis_index("c")
        sid = lax.axis_index("s")

        zero16 = jnp.zeros((16,), jnp.float32)

        for sl in range(s_half):
            slab = cid * s_half + sl
            for h in range(2):
                def zbody(i, _):
                    rows_v[i] = zero16
                    return 0
                lax.fori_loop(0, zrows, zbody, 0)
                for q in range(2):
                    pltpu.sync_copy(
                        rows_v.at[pl.ds(0, zrows)],
                        acc.at[pl.ds(sid * stripe + q * zrows, zrows)])
                @pl.when(sid == 0)
                def _():
                    pltpu.sync_copy(rows_v.at[pl.ds(0, 64)],
                                    acc.at[pl.ds(half, 64)])
                plsc.subcore_barrier()

                base_node = h * half

                def chunk_body(j, _):
                    base = sid * ept + j * _C
                    pltpu.sync_copy(src_hbm.at[pl.ds(base, _C)], idx_v)
                    pltpu.sync_copy(dst_hbm.at[pl.ds(base, _C)], dst_v)

                    def scale_body(i, _):
                        v = idx_v[pl.ds(i * 16, 16)]
                        sidx_v[pl.ds(i * 16, 16)] = v * n_slabs + slab
                        d = dst_v[pl.ds(i * 16, 16)]
                        rel = d - base_node
                        ok = rel.astype(jnp.uint32) < jnp.uint32(half)
                        trash = half + (d & 63)
                        dstm_v[pl.ds(i * 16, 16)] = jnp.where(ok, rel, trash)
                        return 0
                    lax.fori_loop(0, _C // 16, scale_body, 0)

                    pltpu.async_copy(table_hbm.at[sidx_v], rows_v, sem).wait()
                    pltpu.sync_copy(rows_v, acc.at[dstm_v], add=True)
                    return 0
                lax.fori_loop(0, n_chunks, chunk_body, 0)
                plsc.subcore_barrier()

                pltpu.sync_copy(
                    acc.at[pl.ds(sid * stripe, stripe)],
                    out_hbm.at[slab, pl.ds(base_node + sid * stripe, stripe)])
                plsc.subcore_barrier()

    def run(table, src, dst):
        return kern(table, src, dst)

    return run


@functools.lru_cache(maxsize=None)
def _make_seg_sum_scalar(n_nodes, n_edges):
    n_pad = _pad16(n_nodes)
    stripe = n_pad // _NS
    epw = n_edges // (_NS * _NC)
    assert n_edges % (_NS * _NC * _C) == 0
    n_chunks = epw // _C
    zrows = stripe // 4
    assert stripe % 4 == 0 and zrows % 8 == 0

    mesh = plsc.VectorSubcoreMesh(core_axis_name="c", subcore_axis_name="s")

    @functools.partial(
        pl.kernel,
        mesh=mesh,
        compiler_params=pltpu.CompilerParams(use_tc_tiling_on_sc=False),
        out_type=jax.ShapeDtypeStruct((_NC, n_pad), jnp.float32),
        scratch_types=[
            pltpu.VMEM((_C,), jnp.int32),
            pltpu.VMEM((_C,), jnp.int32),
            pltpu.VMEM((_C,), jnp.float32),
            pltpu.VMEM_SHARED((n_pad,), jnp.float32),
            pltpu.SemaphoreType.DMA,
        ],
    )
    def kern(vals_hbm, src_hbm, dst_hbm, out_hbm,
             idx_v, dst_v, vals_v, acc, sem):
        cid = lax.axis_index("c")
        sid = lax.axis_index("s")

        zero16 = jnp.zeros((16,), jnp.float32)

        def zbody(i, _):
            vals_v[pl.ds(i * 16, 16)] = zero16
            return 0
        lax.fori_loop(0, zrows // 16, zbody, 0)
        for q in range(4):
            pltpu.sync_copy(
                vals_v.at[pl.ds(0, zrows)],
                acc.at[pl.ds(sid * stripe + q * zrows, zrows)])
        plsc.subcore_barrier()

        def chunk_body(j, _):
            base = (cid * _NS + sid) * epw + j * _C
            pltpu.sync_copy(src_hbm.at[pl.ds(base, _C)], idx_v)
            pltpu.sync_copy(dst_hbm.at[pl.ds(base, _C)], dst_v)
            pltpu.async_copy(vals_hbm.at[idx_v], vals_v, sem).wait()
            pltpu.sync_copy(vals_v, acc.at[dst_v], add=True)
            return 0
        lax.fori_loop(0, n_chunks, chunk_body, 0)
        plsc.subcore_barrier()

        pltpu.sync_copy(
            acc.at[pl.ds(sid * stripe, stripe)],
            out_hbm.at[cid, pl.ds(sid * stripe, stripe)])

    def run(vals, src, dst):
        return kern(vals, src, dst)

    return run


_BN = 4000


def _k1_body(a1_ref, xp_ref, wl_ref, wr_ref, b_ref, h1_ref, invd_ref):
    a1 = a1_ref[...]
    invd = 1.0 / jnp.maximum(a1[:, 31:32], 1.0)
    agg = jnp.dot(a1, wl_ref[...], preferred_element_type=jnp.float32) * invd
    root = jnp.dot(xp_ref[...], wr_ref[...], preferred_element_type=jnp.float32)
    h1_ref[...] = jnp.maximum(agg + root + b_ref[...], 0.0)
    invd_ref[...] = invd


def _k2_body(a2_ref, h1_ref, invd_ref, wl_ref, wr_ref, b_ref, w3l_ref,
             h2_ref, s3_ref):
    invd = invd_ref[...]
    agg = jnp.dot(a2_ref[...], wl_ref[...], preferred_element_type=jnp.float32)
    root = jnp.dot(h1_ref[...], wr_ref[...], preferred_element_type=jnp.float32)
    h2 = jnp.maximum(agg * invd + root + b_ref[...], 0.0)
    h2_ref[...] = h2
    s3_ref[...] = jnp.dot(h2, w3l_ref[...], preferred_element_type=jnp.float32)


def _k3_body(h2_ref, p_ref, invd_ref, w3r_ref, b_ref, out_ref):
    p = p_ref[...]
    agg = (p[:, 0:1] + p[:, 1:2]) * invd_ref[...]
    root = jnp.dot(h2_ref[...], w3r_ref[...], preferred_element_type=jnp.float32)
    out_ref[...] = agg + root + b_ref[...]


def _row_spec(cols):
    return pl.BlockSpec((_BN, cols), lambda i: (i, 0))


def _full_spec(shape):
    return pl.BlockSpec(shape, lambda i: tuple(0 for _ in shape))


def _dense1(a1, xp, wl, wr, b):
    n = a1.shape[0]
    return pl.pallas_call(
        _k1_body,
        grid=(n // _BN,),
        in_specs=[_row_spec(32), _row_spec(32), _full_spec((32, 128)),
                  _full_spec((32, 128)), _full_spec((1, 128))],
        out_specs=[_row_spec(128), _row_spec(1)],
        out_shape=[jax.ShapeDtypeStruct((n, 128), jnp.float32),
                   jax.ShapeDtypeStruct((n, 1), jnp.float32)],
    )(a1, xp, wl, wr, b)


def _dense2(a2, h1, invd, wl, wr, b, w3l):
    n = a2.shape[0]
    return pl.pallas_call(
        _k2_body,
        grid=(n // _BN,),
        in_specs=[_row_spec(128), _row_spec(128), _row_spec(1),
                  _full_spec((128, 256)), _full_spec((128, 256)),
                  _full_spec((1, 256)), _full_spec((256, 1))],
        out_specs=[_row_spec(256), _row_spec(1)],
        out_shape=[jax.ShapeDtypeStruct((n, 256), jnp.float32),
                   jax.ShapeDtypeStruct((n, 1), jnp.float32)],
    )(a2, h1, invd, wl, wr, b, w3l)


def _dense3(h2, p, invd, w3r, b):
    n = h2.shape[0]
    return pl.pallas_call(
        _k3_body,
        grid=(n // _BN,),
        in_specs=[_row_spec(256), _row_spec(2), _row_spec(1),
                  _full_spec((256, 1)), _full_spec((1, 1))],
        out_specs=_row_spec(1),
        out_shape=jax.ShapeDtypeStruct((n, 1), jnp.float32),
    )(h2, p, invd, w3r, b)


def kernel(x, edge_index, edge_attr, W1_l, b1, W1_r, W2_l, b2, W2_r,
           W3_l, b3, W3_r):
    n, f = x.shape
    e = edge_index.shape[1]
    src = edge_index[0].astype(jnp.int32)
    dst = edge_index[1].astype(jnp.int32)

    xp = jnp.pad(x, ((0, 0), (0, 32 - f)))
    t1 = xp.at[:, 31].set(1.0).reshape(n * 2, 16)
    acc1 = _make_seg_sum_slabs(n, e, 2)(t1, src, dst)
    a1 = acc1[:, :n, :].transpose(1, 0, 2).reshape(n, 32)

    wl1 = jnp.pad(W1_l, ((0, 32 - f), (0, 0)))
    wr1 = jnp.pad(W1_r, ((0, 32 - f), (0, 0)))
    h1, invd = _dense1(a1, xp, wl1, wr1, b1.reshape(1, 128))

    acc2 = _make_seg_sum_slabs(n, e, 8)(h1.reshape(n * 8, 16), src, dst)
    a2 = acc2[:, :n, :].transpose(1, 0, 2).reshape(n, 128)
    h2, s3 = _dense2(a2, h1, invd, W2_l, W2_r, b2.reshape(1, 256), W3_l)

    p3 = _make_seg_sum_scalar(n, e)(s3.reshape(n), src, dst)
    p3t = p3[:, :n].T
    return _dense3(h2, p3t, invd, W3_r, b3.reshape(1, 1))


# speedup vs baseline: 8.1971x; 1.3921x over previous
import functools

import jax
import jax.numpy as jnp
from jax import lax
from jax.experimental import pallas as pl
from jax.experimental.pallas import tpu as pltpu
from jax.experimental.pallas import tpu_sc as plsc

_NC = 2
_NS = 16
_C = 2000


def _pad16(n):
    m = _NS * 32
    return ((n + m - 1) // m) * m


@functools.lru_cache(maxsize=None)
def _make_seg_sum_slabs(n_nodes, n_edges, n_slabs):
    assert n_slabs % 2 == 0
    s_half = n_slabs // 2
    n_pad = _pad16(n_nodes)
    half = n_pad // 2
    stripe = half // _NS
    ept = n_edges // _NS
    assert n_edges % (_NS * _C) == 0
    n_chunks = ept // _C
    zrows = stripe // 2
    acc_rows = half + 64

    mesh = plsc.VectorSubcoreMesh(core_axis_name="c", subcore_axis_name="s")

    @functools.partial(
        pl.kernel,
        mesh=mesh,
        compiler_params=pltpu.CompilerParams(use_tc_tiling_on_sc=False),
        out_type=jax.ShapeDtypeStruct((n_slabs, n_pad, 16), jnp.float32),
        scratch_types=[
            pltpu.VMEM((_C,), jnp.int32),
            pltpu.VMEM((_C,), jnp.int32),
            pltpu.VMEM((_C,), jnp.int32),
            pltpu.VMEM((_C,), jnp.int32),
            pltpu.VMEM((_C,), jnp.int32),
            pltpu.VMEM((_C,), jnp.int32),
            pltpu.VMEM((_C, 16), jnp.float32),
            pltpu.VMEM((_C, 16), jnp.float32),
            pltpu.VMEM_SHARED((acc_rows, 16), jnp.float32),
            pltpu.SemaphoreType.DMA,
            pltpu.SemaphoreType.DMA,
        ],
    )
    def kern(table_hbm, src_hbm, dst_hbm, out_hbm,
             idx_v, sidx0, sidx1, dst_v, dstm0, dstm1, rows0, rows1,
             acc, sem0, sem1):
        cid = lax.axis_index("c")
        sid = lax.axis_index("s")
        sidx = (sidx0, sidx1)
        dstm = (dstm0, dstm1)
        rows = (rows0, rows1)
        sems = (sem0, sem1)

        zero16 = jnp.zeros((16,), jnp.float32)
        n_pairs = n_chunks // 2

        for sl in range(s_half):
            slab = cid * s_half + sl
            for h in range(2):
                def zbody(i, _):
                    rows0[i] = zero16
                    return 0
                lax.fori_loop(0, zrows, zbody, 0)
                for q in range(2):
                    pltpu.sync_copy(
                        rows0.at[pl.ds(0, zrows)],
                        acc.at[pl.ds(sid * stripe + q * zrows, zrows)])
                @pl.when(sid == 0)
                def _():
                    pltpu.sync_copy(rows0.at[pl.ds(0, 64)],
                                    acc.at[pl.ds(half, 64)])
                plsc.subcore_barrier()

                base_node = h * half

                def load_and_gather(j, b):
                    base = sid * ept + j * _C
                    pltpu.sync_copy(src_hbm.at[pl.ds(base, _C)], idx_v)
                    pltpu.sync_copy(dst_hbm.at[pl.ds(base, _C)], dst_v)

                    def scale_body(i, _):
                        v = idx_v[pl.ds(i * 16, 16)]
                        sidx[b][pl.ds(i * 16, 16)] = v * n_slabs + slab
                        d = dst_v[pl.ds(i * 16, 16)]
                        rel = d - base_node
                        ok = rel.astype(jnp.uint32) < jnp.uint32(half)
                        trash = half + (d & 63)
                        dstm[b][pl.ds(i * 16, 16)] = jnp.where(ok, rel, trash)
                        return 0
                    lax.fori_loop(0, _C // 16, scale_body, 0)
                    return pltpu.async_copy(table_hbm.at[sidx[b]],
                                            rows[b], sems[b])

                def scatter(b):
                    pltpu.sync_copy(rows[b], acc.at[dstm[b]], add=True)

                g0 = load_and_gather(0, 0)

                def pair_body(jj, _):
                    c0 = 2 * jj
                    g1 = load_and_gather(c0 + 1, 1)
                    pltpu.make_async_copy(table_hbm.at[sidx[0]],
                                          rows[0], sems[0]).wait()
                    scatter(0)
                    @pl.when(jj + 1 < n_pairs)
                    def _():
                        load_and_gather(c0 + 2, 0)
                    pltpu.make_async_copy(table_hbm.at[sidx[1]],
                                          rows[1], sems[1]).wait()
                    scatter(1)
                    return 0
                lax.fori_loop(0, n_pairs, pair_body, 0)
                plsc.subcore_barrier()

                pltpu.sync_copy(
                    acc.at[pl.ds(sid * stripe, stripe)],
                    out_hbm.at[slab, pl.ds(base_node + sid * stripe, stripe)])
                plsc.subcore_barrier()

    def run(table, src, dst):
        return kern(table, src, dst)

    return run


@functools.lru_cache(maxsize=None)
def _make_seg_sum_scalar(n_nodes, n_edges):
    n_pad = _pad16(n_nodes)
    stripe = n_pad // _NS
    epw = n_edges // (_NS * _NC)
    assert n_edges % (_NS * _NC * _C) == 0
    n_chunks = epw // _C
    zrows = stripe // 4
    assert stripe % 4 == 0 and zrows % 8 == 0

    mesh = plsc.VectorSubcoreMesh(core_axis_name="c", subcore_axis_name="s")

    @functools.partial(
        pl.kernel,
        mesh=mesh,
        compiler_params=pltpu.CompilerParams(use_tc_tiling_on_sc=False),
        out_type=jax.ShapeDtypeStruct((_NC, n_pad), jnp.float32),
        scratch_types=[
            pltpu.VMEM((_C,), jnp.int32),
            pltpu.VMEM((_C,), jnp.int32),
            pltpu.VMEM((_C,), jnp.float32),
            pltpu.VMEM_SHARED((n_pad,), jnp.float32),
            pltpu.SemaphoreType.DMA,
        ],
    )
    def kern(vals_hbm, src_hbm, dst_hbm, out_hbm,
             idx_v, dst_v, vals_v, acc, sem):
        cid = lax.axis_index("c")
        sid = lax.axis_index("s")

        zero16 = jnp.zeros((16,), jnp.float32)

        def zbody(i, _):
            vals_v[pl.ds(i * 16, 16)] = zero16
            return 0
        lax.fori_loop(0, zrows // 16, zbody, 0)
        for q in range(4):
            pltpu.sync_copy(
                vals_v.at[pl.ds(0, zrows)],
                acc.at[pl.ds(sid * stripe + q * zrows, zrows)])
        plsc.subcore_barrier()

        def chunk_body(j, _):
            base = (cid * _NS + sid) * epw + j * _C
            pltpu.sync_copy(src_hbm.at[pl.ds(base, _C)], idx_v)
            pltpu.sync_copy(dst_hbm.at[pl.ds(base, _C)], dst_v)
            pltpu.async_copy(vals_hbm.at[idx_v], vals_v, sem).wait()
            pltpu.sync_copy(vals_v, acc.at[dst_v], add=True)
            return 0
        lax.fori_loop(0, n_chunks, chunk_body, 0)
        plsc.subcore_barrier()

        pltpu.sync_copy(
            acc.at[pl.ds(sid * stripe, stripe)],
            out_hbm.at[cid, pl.ds(sid * stripe, stripe)])

    def run(vals, src, dst):
        return kern(vals, src, dst)

    return run


_BN = 4000


def _k1_body(a1_ref, xp_ref, wl_ref, wr_ref, b_ref, h1_ref, invd_ref):
    a1 = a1_ref[...]
    invd = 1.0 / jnp.maximum(a1[:, 31:32], 1.0)
    agg = jnp.dot(a1, wl_ref[...], preferred_element_type=jnp.float32) * invd
    root = jnp.dot(xp_ref[...], wr_ref[...], preferred_element_type=jnp.float32)
    h1_ref[...] = jnp.maximum(agg + root + b_ref[...], 0.0)
    invd_ref[...] = invd


def _k2_body(a2_ref, h1_ref, invd_ref, wl_ref, wr_ref, b_ref, w3l_ref,
             h2_ref, s3_ref):
    invd = invd_ref[...]
    agg = jnp.dot(a2_ref[...], wl_ref[...], preferred_element_type=jnp.float32)
    root = jnp.dot(h1_ref[...], wr_ref[...], preferred_element_type=jnp.float32)
    h2 = jnp.maximum(agg * invd + root + b_ref[...], 0.0)
    h2_ref[...] = h2
    s3_ref[...] = jnp.dot(h2, w3l_ref[...], preferred_element_type=jnp.float32)


def _k3_body(h2_ref, p_ref, invd_ref, w3r_ref, b_ref, out_ref):
    p = p_ref[...]
    agg = (p[:, 0:1] + p[:, 1:2]) * invd_ref[...]
    root = jnp.dot(h2_ref[...], w3r_ref[...], preferred_element_type=jnp.float32)
    out_ref[...] = agg + root + b_ref[...]


def _row_spec(cols):
    return pl.BlockSpec((_BN, cols), lambda i: (i, 0))


def _full_spec(shape):
    return pl.BlockSpec(shape, lambda i: tuple(0 for _ in shape))


def _dense1(a1, xp, wl, wr, b):
    n = a1.shape[0]
    return pl.pallas_call(
        _k1_body,
        grid=(n // _BN,),
        in_specs=[_row_spec(32), _row_spec(32), _full_spec((32, 128)),
                  _full_spec((32, 128)), _full_spec((1, 128))],
        out_specs=[_row_spec(128), _row_spec(1)],
        out_shape=[jax.ShapeDtypeStruct((n, 128), jnp.float32),
                   jax.ShapeDtypeStruct((n, 1), jnp.float32)],
    )(a1, xp, wl, wr, b)


def _dense2(a2, h1, invd, wl, wr, b, w3l):
    n = a2.shape[0]
    return pl.pallas_call(
        _k2_body,
        grid=(n // _BN,),
        in_specs=[_row_spec(128), _row_spec(128), _row_spec(1),
                  _full_spec((128, 256)), _full_spec((128, 256)),
                  _full_spec((1, 256)), _full_spec((256, 1))],
        out_specs=[_row_spec(256), _row_spec(1)],
        out_shape=[jax.ShapeDtypeStruct((n, 256), jnp.float32),
                   jax.ShapeDtypeStruct((n, 1), jnp.float32)],
    )(a2, h1, invd, wl, wr, b, w3l)


def _dense3(h2, p, invd, w3r, b):
    n = h2.shape[0]
    return pl.pallas_call(
        _k3_body,
        grid=(n // _BN,),
        in_specs=[_row_spec(256), _row_spec(2), _row_spec(1),
                  _full_spec((256, 1)), _full_spec((1, 1))],
        out_specs=_row_spec(1),
        out_shape=jax.ShapeDtypeStruct((n, 1), jnp.float32),
    )(h2, p, invd, w3r, b)


def kernel(x, edge_index, edge_attr, W1_l, b1, W1_r, W2_l, b2, W2_r,
           W3_l, b3, W3_r):
    n, f = x.shape
    e = edge_index.shape[1]
    src = edge_index[0].astype(jnp.int32)
    dst = edge_index[1].astype(jnp.int32)

    xp = jnp.pad(x, ((0, 0), (0, 32 - f)))
    t1 = xp.at[:, 31].set(1.0).reshape(n * 2, 16)
    acc1 = _make_seg_sum_slabs(n, e, 2)(t1, src, dst)
    a1 = acc1[:, :n, :].transpose(1, 0, 2).reshape(n, 32)

    wl1 = jnp.pad(W1_l, ((0, 32 - f), (0, 0)))
    wr1 = jnp.pad(W1_r, ((0, 32 - f), (0, 0)))
    h1, invd = _dense1(a1, xp, wl1, wr1, b1.reshape(1, 128))

    acc2 = _make_seg_sum_slabs(n, e, 8)(h1.reshape(n * 8, 16), src, dst)
    a2 = acc2[:, :n, :].transpose(1, 0, 2).reshape(n, 128)
    h2, s3 = _dense2(a2, h1, invd, W2_l, W2_r, b2.reshape(1, 256), W3_l)

    p3 = _make_seg_sum_scalar(n, e)(s3.reshape(n), src, dst)
    p3t = p3[:, :n].T
    return _dense3(h2, p3t, invd, W3_r, b3.reshape(1, 1))


# speedup vs baseline: 8.6527x; 1.0556x over previous
import functools

import jax
import jax.numpy as jnp
from jax import lax
from jax.experimental import pallas as pl
from jax.experimental.pallas import tpu as pltpu
from jax.experimental.pallas import tpu_sc as plsc

_NC = 2
_NS = 16
_C = 2000


def _pad16(n):
    m = _NS * 32
    return ((n + m - 1) // m) * m


@functools.lru_cache(maxsize=None)
def _make_seg_sum_slabs(n_nodes, n_edges, n_slabs):
    assert n_slabs % 2 == 0
    s_half = n_slabs // 2
    n_pad = _pad16(n_nodes)
    half = n_pad // 2
    stripe = half // _NS
    ept = n_edges // _NS
    assert n_edges % (_NS * _C) == 0
    n_chunks = ept // _C
    zrows = stripe // 2
    acc_rows = half + 64

    mesh = plsc.VectorSubcoreMesh(core_axis_name="c", subcore_axis_name="s")

    @functools.partial(
        pl.kernel,
        mesh=mesh,
        compiler_params=pltpu.CompilerParams(use_tc_tiling_on_sc=False),
        out_type=jax.ShapeDtypeStruct((n_pad, n_slabs, 16), jnp.float32),
        scratch_types=[
            pltpu.VMEM((_C,), jnp.int32),
            pltpu.VMEM((_C,), jnp.int32),
            pltpu.VMEM((_C,), jnp.int32),
            pltpu.VMEM((_C,), jnp.int32),
            pltpu.VMEM((_C,), jnp.int32),
            pltpu.VMEM((_C,), jnp.int32),
            pltpu.VMEM((_C, 16), jnp.float32),
            pltpu.VMEM((_C, 16), jnp.float32),
            pltpu.VMEM_SHARED((acc_rows, 16), jnp.float32),
            pltpu.SemaphoreType.DMA,
            pltpu.SemaphoreType.DMA,
        ],
    )
    def kern(table_hbm, src_hbm, dst_hbm, out_hbm,
             idx_v, sidx0, sidx1, dst_v, dstm0, dstm1, rows0, rows1,
             acc, sem0, sem1):
        cid = lax.axis_index("c")
        sid = lax.axis_index("s")
        sidx = (sidx0, sidx1)
        dstm = (dstm0, dstm1)
        rows = (rows0, rows1)
        sems = (sem0, sem1)

        zero16 = jnp.zeros((16,), jnp.float32)
        n_pairs = n_chunks // 2

        for sl in range(s_half):
            slab = cid * s_half + sl
            for h in range(2):
                def zbody(i, _):
                    rows0[i] = zero16
                    return 0
                lax.fori_loop(0, zrows, zbody, 0)
                for q in range(2):
                    pltpu.sync_copy(
                        rows0.at[pl.ds(0, zrows)],
                        acc.at[pl.ds(sid * stripe + q * zrows, zrows)])
                @pl.when(sid == 0)
                def _():
                    pltpu.sync_copy(rows0.at[pl.ds(0, 64)],
                                    acc.at[pl.ds(half, 64)])
                plsc.subcore_barrier()

                base_node = h * half

                def load_and_gather(j, b):
                    base = sid * ept + j * _C
                    pltpu.sync_copy(src_hbm.at[pl.ds(base, _C)], idx_v)
                    pltpu.sync_copy(dst_hbm.at[pl.ds(base, _C)], dst_v)

                    def scale_body(i, _):
                        v = idx_v[pl.ds(i * 16, 16)]
                        sidx[b][pl.ds(i * 16, 16)] = v * n_slabs + slab
                        d = dst_v[pl.ds(i * 16, 16)]
                        rel = d - base_node
                        ok = rel.astype(jnp.uint32) < jnp.uint32(half)
                        trash = half + (d & 63)
                        dstm[b][pl.ds(i * 16, 16)] = jnp.where(ok, rel, trash)
                        return 0
                    lax.fori_loop(0, _C // 16, scale_body, 0)
                    return pltpu.async_copy(table_hbm.at[sidx[b]],
                                            rows[b], sems[b])

                def scatter(b):
                    pltpu.sync_copy(rows[b], acc.at[dstm[b]], add=True)

                g0 = load_and_gather(0, 0)

                def pair_body(jj, _):
                    c0 = 2 * jj
                    g1 = load_and_gather(c0 + 1, 1)
                    pltpu.make_async_copy(table_hbm.at[sidx[0]],
                                          rows[0], sems[0]).wait()
                    scatter(0)
                    @pl.when(jj + 1 < n_pairs)
                    def _():
                        load_and_gather(c0 + 2, 0)
                    pltpu.make_async_copy(table_hbm.at[sidx[1]],
                                          rows[1], sems[1]).wait()
                    scatter(1)
                    return 0
                lax.fori_loop(0, n_pairs, pair_body, 0)
                plsc.subcore_barrier()

                pltpu.sync_copy(
                    acc.at[pl.ds(sid * stripe, stripe)],
                    out_hbm.at[pl.ds(base_node + sid * stripe, stripe), slab])
                plsc.subcore_barrier()

    def run(table, src, dst):
        return kern(table, src, dst)

    return run


@functools.lru_cache(maxsize=None)
def _make_seg_sum_scalar(n_nodes, n_edges):
    n_pad = _pad16(n_nodes)
    stripe = n_pad // _NS
    epw = n_edges // (_NS * _NC)
    assert n_edges % (_NS * _NC * _C) == 0
    n_chunks = epw // _C
    zrows = stripe // 4
    assert stripe % 4 == 0 and zrows % 8 == 0

    mesh = plsc.VectorSubcoreMesh(core_axis_name="c", subcore_axis_name="s")

    @functools.partial(
        pl.kernel,
        mesh=mesh,
        compiler_params=pltpu.CompilerParams(use_tc_tiling_on_sc=False),
        out_type=jax.ShapeDtypeStruct((_NC, n_pad), jnp.float32),
        scratch_types=[
            pltpu.VMEM((_C,), jnp.int32),
            pltpu.VMEM((_C,), jnp.int32),
            pltpu.VMEM((_C,), jnp.float32),
            pltpu.VMEM_SHARED((n_pad,), jnp.float32),
            pltpu.SemaphoreType.DMA,
        ],
    )
    def kern(vals_hbm, src_hbm, dst_hbm, out_hbm,
             idx_v, dst_v, vals_v, acc, sem):
        cid = lax.axis_index("c")
        sid = lax.axis_index("s")

        zero16 = jnp.zeros((16,), jnp.float32)

        def zbody(i, _):
            vals_v[pl.ds(i * 16, 16)] = zero16
            return 0
        lax.fori_loop(0, zrows // 16, zbody, 0)
        for q in range(4):
            pltpu.sync_copy(
                vals_v.at[pl.ds(0, zrows)],
                acc.at[pl.ds(sid * stripe + q * zrows, zrows)])
        plsc.subcore_barrier()

        def chunk_body(j, _):
            base = (cid * _NS + sid) * epw + j * _C
            pltpu.sync_copy(src_hbm.at[pl.ds(base, _C)], idx_v)
            pltpu.sync_copy(dst_hbm.at[pl.ds(base, _C)], dst_v)
            pltpu.async_copy(vals_hbm.at[idx_v], vals_v, sem).wait()
            pltpu.sync_copy(vals_v, acc.at[dst_v], add=True)
            return 0
        lax.fori_loop(0, n_chunks, chunk_body, 0)
        plsc.subcore_barrier()

        pltpu.sync_copy(
            acc.at[pl.ds(sid * stripe, stripe)],
            out_hbm.at[cid, pl.ds(sid * stripe, stripe)])

    def run(vals, src, dst):
        return kern(vals, src, dst)

    return run


_BN = 6272


def _k1_body(a1_ref, xp_ref, wl_ref, wr_ref, b_ref, h1_ref, invd_ref):
    a1 = a1_ref[...]
    invd = 1.0 / jnp.maximum(a1[:, 31:32], 1.0)
    agg = jnp.dot(a1, wl_ref[...], preferred_element_type=jnp.float32) * invd
    root = jnp.dot(xp_ref[...], wr_ref[...], preferred_element_type=jnp.float32)
    h1_ref[...] = jnp.maximum(agg + root + b_ref[...], 0.0)
    invd_ref[...] = invd


def _k2_body(a2_ref, h1_ref, invd_ref, wl_ref, wr_ref, b_ref, w3l_ref,
             h2_ref, s3_ref):
    invd = invd_ref[...]
    agg = jnp.dot(a2_ref[...], wl_ref[...], preferred_element_type=jnp.float32)
    root = jnp.dot(h1_ref[...], wr_ref[...], preferred_element_type=jnp.float32)
    h2 = jnp.maximum(agg * invd + root + b_ref[...], 0.0)
    h2_ref[...] = h2
    s3_ref[...] = jnp.dot(h2, w3l_ref[...], preferred_element_type=jnp.float32)


def _k3_body(h2_ref, p_ref, invd_ref, w3r_ref, b_ref, out_ref):
    p = p_ref[...]
    agg = (p[:, 0:1] + p[:, 1:2]) * invd_ref[...]
    root = jnp.dot(h2_ref[...], w3r_ref[...], preferred_element_type=jnp.float32)
    out_ref[...] = agg + root + b_ref[...]


def _row_spec(cols):
    return pl.BlockSpec((_BN, cols), lambda i: (i, 0))


def _full_spec(shape):
    return pl.BlockSpec(shape, lambda i: tuple(0 for _ in shape))


def _dense1(a1, xp, wl, wr, b):
    n = a1.shape[0]
    return pl.pallas_call(
        _k1_body,
        grid=(n // _BN,),
        in_specs=[_row_spec(32), _row_spec(32), _full_spec((32, 128)),
                  _full_spec((32, 128)), _full_spec((1, 128))],
        out_specs=[_row_spec(128), _row_spec(1)],
        out_shape=[jax.ShapeDtypeStruct((n, 128), jnp.float32),
                   jax.ShapeDtypeStruct((n, 1), jnp.float32)],
    )(a1, xp, wl, wr, b)


def _dense2(a2, h1, invd, wl, wr, b, w3l):
    n = a2.shape[0]
    return pl.pallas_call(
        _k2_body,
        grid=(n // _BN,),
        in_specs=[_row_spec(128), _row_spec(128), _row_spec(1),
                  _full_spec((128, 256)), _full_spec((128, 256)),
                  _full_spec((1, 256)), _full_spec((256, 1))],
        out_specs=[_row_spec(256), _row_spec(1)],
        out_shape=[jax.ShapeDtypeStruct((n, 256), jnp.float32),
                   jax.ShapeDtypeStruct((n, 1), jnp.float32)],
    )(a2, h1, invd, wl, wr, b, w3l)


def _dense3(h2, p, invd, w3r, b):
    n = h2.shape[0]
    return pl.pallas_call(
        _k3_body,
        grid=(n // _BN,),
        in_specs=[_row_spec(256), _row_spec(2), _row_spec(1),
                  _full_spec((256, 1)), _full_spec((1, 1))],
        out_specs=_row_spec(1),
        out_shape=jax.ShapeDtypeStruct((n, 1), jnp.float32),
    )(h2, p, invd, w3r, b)


def kernel(x, edge_index, edge_attr, W1_l, b1, W1_r, W2_l, b2, W2_r,
           W3_l, b3, W3_r):
    n, f = x.shape
    n_pad = _pad16(n)
    e = edge_index.shape[1]
    src = edge_index[0].astype(jnp.int32)
    dst = edge_index[1].astype(jnp.int32)

    t1 = jnp.pad(x, ((0, 0), (0, 32 - f)),
                 constant_values=((0.0, 0.0), (0.0, 1.0))).reshape(n * 2, 16)
    xp = jnp.pad(x, ((0, n_pad - n), (0, 32 - f)))
    acc1 = _make_seg_sum_slabs(n, e, 2)(t1, src, dst)
    a1 = acc1.reshape(n_pad, 32)

    wl1 = jnp.pad(W1_l, ((0, 32 - f), (0, 0)))
    wr1 = jnp.pad(W1_r, ((0, 32 - f), (0, 0)))
    h1, invd = _dense1(a1, xp, wl1, wr1, b1.reshape(1, 128))

    acc2 = _make_seg_sum_slabs(n, e, 8)(h1.reshape(n_pad * 8, 16), src, dst)
    a2 = acc2.reshape(n_pad, 128)
    h2, s3 = _dense2(a2, h1, invd, W2_l, W2_r, b2.reshape(1, 256), W3_l)

    p3 = _make_seg_sum_scalar(n, e)(s3.reshape(n_pad), src, dst)
    p3t = p3.T
    out = _dense3(h2, p3t, invd, W3_r, b3.reshape(1, 1))
    return out[:n]


# speedup vs baseline: 10.7778x; 1.2456x over previous
import functools

import jax
import jax.numpy as jnp
from jax import lax
from jax.experimental import pallas as pl
from jax.experimental.pallas import tpu as pltpu
from jax.experimental.pallas import tpu_sc as plsc

_NC = 2
_NS = 16
_C = 2000


def _pad16(n):
    m = _NS * 32
    return ((n + m - 1) // m) * m


@functools.lru_cache(maxsize=None)
def _make_seg_sum_slabs(n_nodes, n_edges, n_slabs):
    assert n_slabs % 2 == 0
    s_half = n_slabs // 2
    n_pad = _pad16(n_nodes)
    half = n_pad // 2
    stripe = half // _NS
    ept = n_edges // _NS
    assert n_edges % (_NS * _C) == 0
    n_chunks = ept // _C
    zrows = stripe // 2
    acc_rows = half + 64

    mesh = plsc.VectorSubcoreMesh(core_axis_name="c", subcore_axis_name="s")

    @functools.partial(
        pl.kernel,
        mesh=mesh,
        compiler_params=pltpu.CompilerParams(use_tc_tiling_on_sc=False),
        out_type=jax.ShapeDtypeStruct((n_pad, 16 * n_slabs), jnp.float32),
        scratch_types=[
            pltpu.VMEM((_C,), jnp.int32),
            pltpu.VMEM((_C,), jnp.int32),
            pltpu.VMEM((_C,), jnp.int32),
            pltpu.VMEM((_C,), jnp.int32),
            pltpu.VMEM((_C,), jnp.int32),
            pltpu.VMEM((_C,), jnp.int32),
            pltpu.VMEM((_C, 16), jnp.float32),
            pltpu.VMEM((_C, 16), jnp.float32),
            pltpu.VMEM_SHARED((acc_rows, 16), jnp.float32),
            pltpu.SemaphoreType.DMA,
            pltpu.SemaphoreType.DMA,
        ],
    )
    def kern(table_hbm, src_hbm, dst_hbm, out_hbm,
             idx_v, sidx0, sidx1, dst_v, dstm0, dstm1, rows0, rows1,
             acc, sem0, sem1):
        cid = lax.axis_index("c")
        sid = lax.axis_index("s")
        sidx = (sidx0, sidx1)
        dstm = (dstm0, dstm1)
        rows = (rows0, rows1)
        sems = (sem0, sem1)

        zero16 = jnp.zeros((16,), jnp.float32)
        n_pairs = n_chunks // 2

        for sl in range(s_half):
            slab = cid * s_half + sl
            for h in range(2):
                def zbody(i, _):
                    rows0[i] = zero16
                    return 0
                lax.fori_loop(0, zrows, zbody, 0)
                for q in range(2):
                    pltpu.sync_copy(
                        rows0.at[pl.ds(0, zrows)],
                        acc.at[pl.ds(sid * stripe + q * zrows, zrows)])
                @pl.when(sid == 0)
                def _():
                    pltpu.sync_copy(rows0.at[pl.ds(0, 64)],
                                    acc.at[pl.ds(half, 64)])
                plsc.subcore_barrier()

                base_node = h * half

                def load_and_gather(j, b):
                    base = sid * ept + j * _C
                    pltpu.sync_copy(src_hbm.at[pl.ds(base, _C)], idx_v)
                    pltpu.sync_copy(dst_hbm.at[pl.ds(base, _C)], dst_v)

                    def scale_body(i, _):
                        v = idx_v[pl.ds(i * 16, 16)]
                        sidx[b][pl.ds(i * 16, 16)] = v * n_slabs + slab
                        d = dst_v[pl.ds(i * 16, 16)]
                        rel = d - base_node
                        ok = rel.astype(jnp.uint32) < jnp.uint32(half)
                        trash = half + (d & 63)
                        dstm[b][pl.ds(i * 16, 16)] = jnp.where(ok, rel, trash)
                        return 0
                    lax.fori_loop(0, _C // 16, scale_body, 0)
                    return pltpu.async_copy(table_hbm.at[sidx[b]],
                                            rows[b], sems[b])

                def scatter(b):
                    pltpu.sync_copy(rows[b], acc.at[dstm[b]], add=True)

                g0 = load_and_gather(0, 0)

                def pair_body(jj, _):
                    c0 = 2 * jj
                    g1 = load_and_gather(c0 + 1, 1)
                    pltpu.make_async_copy(table_hbm.at[sidx[0]],
                                          rows[0], sems[0]).wait()
                    scatter(0)
                    @pl.when(jj + 1 < n_pairs)
                    def _():
                        load_and_gather(c0 + 2, 0)
                    pltpu.make_async_copy(table_hbm.at[sidx[1]],
                                          rows[1], sems[1]).wait()
                    scatter(1)
                    return 0
                lax.fori_loop(0, n_pairs, pair_body, 0)
                plsc.subcore_barrier()

                pltpu.sync_copy(
                    acc.at[pl.ds(sid * stripe, stripe)],
                    out_hbm.at[pl.ds(base_node + sid * stripe, stripe),
                               pl.ds(slab * 16, 16)])
                plsc.subcore_barrier()

    def run(table, src, dst):
        return kern(table, src, dst)

    return run


@functools.lru_cache(maxsize=None)
def _make_seg_sum_scalar(n_nodes, n_edges):
    n_pad = _pad16(n_nodes)
    stripe = n_pad // _NS
    epw = n_edges // (_NS * _NC)
    assert n_edges % (_NS * _NC * _C) == 0
    n_chunks = epw // _C
    zrows = stripe // 4
    assert stripe % 4 == 0 and zrows % 8 == 0

    mesh = plsc.VectorSubcoreMesh(core_axis_name="c", subcore_axis_name="s")

    @functools.partial(
        pl.kernel,
        mesh=mesh,
        compiler_params=pltpu.CompilerParams(use_tc_tiling_on_sc=False),
        out_type=jax.ShapeDtypeStruct((_NC, n_pad), jnp.float32),
        scratch_types=[
            pltpu.VMEM((_C,), jnp.int32),
            pltpu.VMEM((_C,), jnp.int32),
            pltpu.VMEM((_C,), jnp.float32),
            pltpu.VMEM_SHARED((n_pad,), jnp.float32),
            pltpu.SemaphoreType.DMA,
        ],
    )
    def kern(vals_hbm, src_hbm, dst_hbm, out_hbm,
             idx_v, dst_v, vals_v, acc, sem):
        cid = lax.axis_index("c")
        sid = lax.axis_index("s")

        zero16 = jnp.zeros((16,), jnp.float32)

        def zbody(i, _):
            vals_v[pl.ds(i * 16, 16)] = zero16
            return 0
        lax.fori_loop(0, zrows // 16, zbody, 0)
        for q in range(4):
            pltpu.sync_copy(
                vals_v.at[pl.ds(0, zrows)],
                acc.at[pl.ds(sid * stripe + q * zrows, zrows)])
        plsc.subcore_barrier()

        def chunk_body(j, _):
            base = (cid * _NS + sid) * epw + j * _C
            pltpu.sync_copy(src_hbm.at[pl.ds(base, _C)], idx_v)
            pltpu.sync_copy(dst_hbm.at[pl.ds(base, _C)], dst_v)
            pltpu.async_copy(vals_hbm.at[idx_v], vals_v, sem).wait()
            pltpu.sync_copy(vals_v, acc.at[dst_v], add=True)
            return 0
        lax.fori_loop(0, n_chunks, chunk_body, 0)
        plsc.subcore_barrier()

        pltpu.sync_copy(
            acc.at[pl.ds(sid * stripe, stripe)],
            out_hbm.at[cid, pl.ds(sid * stripe, stripe)])

    def run(vals, src, dst):
        return kern(vals, src, dst)

    return run


_BN = 6272


def _k1_body(a1_ref, xp_ref, wl_ref, wr_ref, b_ref, h1_ref, invd_ref):
    a1 = a1_ref[...]
    invd = 1.0 / jnp.maximum(a1[:, 31:32], 1.0)
    agg = jnp.dot(a1, wl_ref[...], preferred_element_type=jnp.float32) * invd
    root = jnp.dot(xp_ref[...], wr_ref[...], preferred_element_type=jnp.float32)
    h1_ref[...] = jnp.maximum(agg + root + b_ref[...], 0.0)
    invd_ref[...] = invd


def _k2_body(a2_ref, h1_ref, invd_ref, wl_ref, wr_ref, b_ref, w3l_ref,
             h2_ref, s3_ref):
    invd = invd_ref[...]
    agg = jnp.dot(a2_ref[...], wl_ref[...], preferred_element_type=jnp.float32)
    root = jnp.dot(h1_ref[...], wr_ref[...], preferred_element_type=jnp.float32)
    h2 = jnp.maximum(agg * invd + root + b_ref[...], 0.0)
    h2_ref[...] = h2
    s3_ref[...] = jnp.dot(h2, w3l_ref[...], preferred_element_type=jnp.float32)


def _k3_body(h2_ref, p_ref, invd_ref, w3r_ref, b_ref, out_ref):
    p = p_ref[...]
    agg = (p[:, 0:1] + p[:, 1:2]) * invd_ref[...]
    root = jnp.dot(h2_ref[...], w3r_ref[...], preferred_element_type=jnp.float32)
    out_ref[...] = agg + root + b_ref[...]


def _row_spec(cols):
    return pl.BlockSpec((_BN, cols), lambda i: (i, 0))


def _full_spec(shape):
    return pl.BlockSpec(shape, lambda i: tuple(0 for _ in shape))


def _dense1(a1, xp, wl, wr, b):
    n = a1.shape[0]
    return pl.pallas_call(
        _k1_body,
        grid=(n // _BN,),
        in_specs=[_row_spec(32), _row_spec(32), _full_spec((32, 128)),
                  _full_spec((32, 128)), _full_spec((1, 128))],
        out_specs=[_row_spec(128), _row_spec(1)],
        out_shape=[jax.ShapeDtypeStruct((n, 128), jnp.float32),
                   jax.ShapeDtypeStruct((n, 1), jnp.float32)],
    )(a1, xp, wl, wr, b)


def _dense2(a2, h1, invd, wl, wr, b, w3l):
    n = a2.shape[0]
    return pl.pallas_call(
        _k2_body,
        grid=(n // _BN,),
        in_specs=[_row_spec(128), _row_spec(128), _row_spec(1),
                  _full_spec((128, 256)), _full_spec((128, 256)),
                  _full_spec((1, 256)), _full_spec((256, 1))],
        out_specs=[_row_spec(256), _row_spec(1)],
        out_shape=[jax.ShapeDtypeStruct((n, 256), jnp.float32),
                   jax.ShapeDtypeStruct((n, 1), jnp.float32)],
    )(a2, h1, invd, wl, wr, b, w3l)


def _dense3(h2, p, invd, w3r, b):
    n = h2.shape[0]
    return pl.pallas_call(
        _k3_body,
        grid=(n // _BN,),
        in_specs=[_row_spec(256), _row_spec(2), _row_spec(1),
                  _full_spec((256, 1)), _full_spec((1, 1))],
        out_specs=_row_spec(1),
        out_shape=jax.ShapeDtypeStruct((n, 1), jnp.float32),
    )(h2, p, invd, w3r, b)


def kernel(x, edge_index, edge_attr, W1_l, b1, W1_r, W2_l, b2, W2_r,
           W3_l, b3, W3_r):
    n, f = x.shape
    n_pad = _pad16(n)
    e = edge_index.shape[1]
    src = edge_index[0].astype(jnp.int32)
    dst = edge_index[1].astype(jnp.int32)

    t1 = jnp.pad(x, ((0, 0), (0, 32 - f)),
                 constant_values=((0.0, 0.0), (0.0, 1.0))).reshape(n * 2, 16)
    xp = jnp.pad(x, ((0, n_pad - n), (0, 32 - f)))
    a1 = _make_seg_sum_slabs(n, e, 2)(t1, src, dst)

    wl1 = jnp.pad(W1_l, ((0, 32 - f), (0, 0)))
    wr1 = jnp.pad(W1_r, ((0, 32 - f), (0, 0)))
    h1, invd = _dense1(a1, xp, wl1, wr1, b1.reshape(1, 128))

    a2 = _make_seg_sum_slabs(n, e, 8)(h1.reshape(n_pad * 8, 16), src, dst)
    h2, s3 = _dense2(a2, h1, invd, W2_l, W2_r, b2.reshape(1, 256), W3_l)

    p3 = _make_seg_sum_scalar(n, e)(s3.reshape(n_pad), src, dst)
    p3t = p3.T
    out = _dense3(h2, p3t, invd, W3_r, b3.reshape(1, 1))
    return out[:n]
